# Initial kernel scaffold; baseline (speedup 1.0000x reference)
import jax
import jax.numpy as jnp
from jax.experimental import pallas as pl


def kernel(sequence, table):
    raise NotImplementedError("write your pallas kernel here")


# speedup vs baseline: 3.8638x; 3.8638x over previous
import functools

import numpy as np
import jax
import jax.numpy as jnp
from jax import lax
from jax.experimental import pallas as pl
from jax.experimental.pallas import tpu as pltpu
from jax.experimental.pallas import tpu_sc as plsc

NC, NS, L = 2, 16, 16
NW = NC * NS
ED = 10
GRP = 16

_PAT = np.array([[(v * L + l) // ED for l in range(L)] for v in range(ED)], np.int32)
_DIM = np.array([[(v * L + l) % ED for l in range(L)] for v in range(ED)], np.int32)


@functools.partial(jax.jit, static_argnames=("chunk",))
def _sc_embed(seq_flat, table, *, chunk):
    n = seq_flat.shape[0]
    nvocab = table.shape[0]
    per_w = n // NW
    n_chunks = per_w // chunk
    assert n == per_w * NW and per_w == n_chunks * chunk and chunk % GRP == 0

    mesh = plsc.VectorSubcoreMesh(
        core_axis_name="c", subcore_axis_name="s", num_cores=NC, num_subcores=NS
    )

    @functools.partial(
        pl.kernel,
        out_type=jax.ShapeDtypeStruct((n * ED,), jnp.float32),
        mesh=mesh,
        compiler_params=pltpu.CompilerParams(needs_layout_passes=False),
        scratch_types=[
            pltpu.VMEM((nvocab, ED), jnp.float32),
            pltpu.VMEM((chunk,), jnp.int32),
            pltpu.VMEM((chunk * ED,), jnp.float32),
        ],
    )
    def run(seq_hbm, tab_hbm, out_hbm, tab_v, seq_v, out_v):
        wid = lax.axis_index("s") * NC + lax.axis_index("c")
        base = wid * per_w
        pltpu.sync_copy(tab_hbm, tab_v)
        lane = lax.iota(jnp.int32, L)
        pats = [lax.div(lane + v * L, ED) for v in range(ED)]
        dims = [lax.rem(lane + v * L, ED) for v in range(ED)]

        for c in range(n_chunks):
            off = base + c * chunk
            pltpu.sync_copy(seq_hbm.at[pl.ds(off, chunk)], seq_v)

            def body(g, carry):
                p0 = g * GRP
                for v in range(ED):
                    sv = plsc.load_gather(seq_v, [pats[v] + p0])
                    row = plsc.load_gather(tab_v, [sv, dims[v]])
                    out_v[pl.ds(g * (GRP * ED) + v * L, L)] = row
                return carry

            lax.fori_loop(0, chunk // GRP, body, 0)
            pltpu.sync_copy(out_v, out_hbm.at[pl.ds(off * ED, chunk * ED)])

    return run(seq_flat, table)


def kernel(sequence, table):
    b, s = sequence.shape
    v, d = table.shape
    assert d == ED
    seq_flat = sequence.reshape(-1).astype(jnp.int32)
    out_flat = _sc_embed(seq_flat, table.astype(jnp.float32), chunk=4096)
    return out_flat.reshape(b, s, d)
